# baseline (speedup 1.0000x reference)
import functools

import jax
import jax.numpy as jnp
from jax import lax
from jax.experimental import pallas as pl
from jax.experimental.pallas import tpu as pltpu
from jax.experimental.pallas import tpu_sc as plsc

_NC = 2
_NS = 16
_NW = _NC * _NS
_F = 4
_CE = 2000
_UN = 10


def _mesh():
    return plsc.VectorSubcoreMesh(
        core_axis_name="c", subcore_axis_name="s",
        num_cores=_NC, num_subcores=_NS)


def _sc_degree(dst_flat, ew_flat, n_nodes):
    e_total = dst_flat.shape[0]
    epw = e_total // _NW
    nvec = epw // 16
    nz = n_nodes // 16

    @functools.partial(
        pl.kernel,
        out_type=jax.ShapeDtypeStruct((_NW, n_nodes), jnp.float32),
        mesh=_mesh(),
        compiler_params=pltpu.CompilerParams(needs_layout_passes=False),
        scratch_types=[
            pltpu.VMEM((epw,), jnp.int32),
            pltpu.VMEM((epw,), jnp.float32),
            pltpu.VMEM((n_nodes,), jnp.float32),
        ],
    )
    def deg_kernel(dst_hbm, ew_hbm, out_hbm, idx_v, w_v, deg_v):
        c = lax.axis_index("c")
        s = lax.axis_index("s")
        t = c * _NS + s
        pltpu.sync_copy(dst_hbm.at[pl.ds(t * epw, epw)], idx_v)
        pltpu.sync_copy(ew_hbm.at[pl.ds(t * epw, epw)], w_v)
        zeros = jnp.zeros((16,), jnp.float32)

        def zbody(i, carry):
            deg_v[pl.ds(i * 16, 16)] = zeros
            return carry
        lax.fori_loop(0, nz, zbody, 0)

        @plsc.parallel_loop(0, nvec, step=1, unroll=5)
        def _(i):
            idx = idx_v[pl.ds(i * 16, 16)]
            w = w_v[pl.ds(i * 16, 16)]
            plsc.addupdate_scatter(deg_v, [idx], w)
        pltpu.sync_copy(deg_v, out_hbm.at[t])

    return deg_kernel(dst_flat, ew_flat)


def _sc_propagate(table_t, epk, ew, n_nodes):
    h = table_t.shape[0]
    e_total = epk.shape[0]
    owners = h // _F
    es = _NS // owners
    npart = _NC * es
    ept = e_total // npart
    nchunk = ept // _CE
    nvec = _CE // 16
    nz = n_nodes // 16
    assert ept % _CE == 0 and _NS % owners == 0

    @functools.partial(
        pl.kernel,
        out_type=jax.ShapeDtypeStruct((npart, h, n_nodes), jnp.float32),
        mesh=_mesh(),
        compiler_params=pltpu.CompilerParams(needs_layout_passes=False,
                                             use_tc_tiling_on_sc=False),
        scratch_types=(
            [pltpu.VMEM((n_nodes,), jnp.float32) for _ in range(2 * _F)]
            + [pltpu.VMEM((2 * _CE,), jnp.int32),
               pltpu.VMEM((2 * _CE,), jnp.float32),
               pltpu.SemaphoreType.DMA]
        ),
    )
    def prop_kernel(tbl_hbm, epk_hbm, ew_hbm, out_hbm, *rest):
        tbl = rest[:_F]
        acc = rest[_F:2 * _F]
        epk_v = rest[2 * _F]
        ew_v = rest[2 * _F + 1]
        sem = rest[2 * _F + 2]
        c = lax.axis_index("c")
        s = lax.axis_index("s")
        o = lax.rem(s, owners)
        sub = lax.div(s, owners)
        part = c * es + sub
        fo = o * _F
        eb = part * ept

        def stage_start(k):
            off = eb + k * _CE
            slot = lax.rem(k, 2) * _CE
            pltpu.async_copy(epk_hbm.at[pl.ds(off, _CE)],
                             epk_v.at[pl.ds(slot, _CE)], sem)
            pltpu.async_copy(ew_hbm.at[pl.ds(off, _CE)],
                             ew_v.at[pl.ds(slot, _CE)], sem)

        def stage_wait(k):
            off = eb + k * _CE
            slot = lax.rem(k, 2) * _CE
            pltpu.make_async_copy(epk_hbm.at[pl.ds(off, _CE)],
                                  epk_v.at[pl.ds(slot, _CE)], sem).wait()
            pltpu.make_async_copy(ew_hbm.at[pl.ds(off, _CE)],
                                  ew_v.at[pl.ds(slot, _CE)], sem).wait()

        stage_start(0)
        for f in range(_F):
            pltpu.sync_copy(tbl_hbm.at[fo + f], tbl[f])
        zeros = jnp.zeros((16,), jnp.float32)

        def zbody(i, carry):
            for f in range(_F):
                acc[f][pl.ds(i * 16, 16)] = zeros
            return carry
        lax.fori_loop(0, nz, zbody, 0)

        def chunk(k, carry):
            stage_wait(k)

            @pl.when(k + 1 < nchunk)
            def _():
                stage_start(k + 1)
            base = lax.rem(k, 2) * _CE

            @plsc.parallel_loop(0, nvec, step=1, unroll=_UN)
            def _(v):
                off = base + v * 16
                pk = epk_v[pl.ds(off, 16)]
                w = ew_v[pl.ds(off, 16)]
                src16 = lax.bitwise_and(pk, 0xFFFF)
                dst16 = lax.shift_right_logical(pk, 16)
                for f in range(_F):
                    vals = plsc.load_gather(tbl[f], [src16])
                    plsc.addupdate_scatter(acc[f], [dst16], vals * w)
            return carry
        lax.fori_loop(0, nchunk, chunk, 0)

        for f in range(_F):
            pltpu.sync_copy(acc[f], out_hbm.at[part, fo + f])

    return prop_kernel(table_t, epk, ew)


def _tc_mm(x, w1, edge_index):
    n = x.shape[0]
    h1 = w1.shape[1]
    e_total = edge_index.shape[1]

    def body(x_ref, w_ref, ei_ref, s_ref, epk_ref):
        s_ref[...] = lax.dot_general(w_ref[...], x_ref[...],
                                     (((0,), (1,)), ((), ())),
                                     preferred_element_type=jnp.float32)
        epk_ref[...] = jnp.bitwise_or(ei_ref[0],
                                      jnp.left_shift(ei_ref[1], 16))

    return pl.pallas_call(
        body,
        out_shape=(jax.ShapeDtypeStruct((h1, n), jnp.float32),
                   jax.ShapeDtypeStruct((e_total,), jnp.int32)),
    )(x, w1, edge_index)


def _tc_scale(mmt, degp):
    h1, n = mmt.shape

    def body(m_ref, dp_ref, s_ref, dinv_ref):
        deg = jnp.sum(dp_ref[...], axis=0) + 1.0
        dinv = lax.rsqrt(deg)[None, :]
        dinv_ref[...] = dinv
        s_ref[...] = m_ref[...] * dinv

    return pl.pallas_call(
        body,
        out_shape=(jax.ShapeDtypeStruct((h1, n), jnp.float32),
                   jax.ShapeDtypeStruct((1, n), jnp.float32)),
    )(mmt, degp)


def _tc_mid(acc_t, s1_t, dinv, b1, w2):
    h2 = w2.shape[1]
    n = s1_t.shape[1]

    def body(a_ref, s_ref, di_ref, b_ref, w_ref, o_ref):
        hpre = (jnp.sum(a_ref[...], axis=0) + s_ref[...]) * di_ref[...]
        hh = jnp.maximum(hpre + b_ref[...][:, None], 0.0)
        st = lax.dot_general(w_ref[...], hh, (((0,), (0,)), ((), ())),
                             preferred_element_type=jnp.float32)
        o_ref[...] = st * di_ref[...]

    return pl.pallas_call(
        body,
        out_shape=jax.ShapeDtypeStruct((h2, n), jnp.float32),
    )(acc_t, s1_t, dinv, b1, w2)


def _tc_final(acc_t, s2_t, dinv, b2, wl, bl):
    n = s2_t.shape[1]

    def body(a_ref, s_ref, di_ref, b_ref, wl_ref, bl_ref, o_ref):
        hpre = (jnp.sum(a_ref[...], axis=0) + s_ref[...]) * di_ref[...]
        hh = jnp.maximum(hpre + b_ref[...][:, None], 0.0)
        ot = lax.dot_general(wl_ref[...], hh, (((0,), (0,)), ((), ())),
                             preferred_element_type=jnp.float32)
        o_ref[...] = ot + bl_ref[...][:, None]

    return pl.pallas_call(
        body,
        out_shape=jax.ShapeDtypeStruct((1, n), jnp.float32),
    )(acc_t, s2_t, dinv, b2, wl, bl)


def kernel(x, edge_index, edge_weight, W1, b1, W2, b2, Wl, bl):
    n = x.shape[0]
    dst = edge_index[1]

    degp = _sc_degree(dst, edge_weight, n)
    mmt, epk = _tc_mm(x, W1, edge_index)
    s1t, dinv = _tc_scale(mmt, degp)
    acc1 = _sc_propagate(s1t, epk, edge_weight, n)
    s2t = _tc_mid(acc1, s1t, dinv, b1, W2)
    acc2 = _sc_propagate(s2t, epk, edge_weight, n)
    out = _tc_final(acc2, s2t, dinv, b2, Wl, bl)
    return out[0]


# speedup vs baseline: 1.0767x; 1.0767x over previous
import functools

import jax
import jax.numpy as jnp
from jax import lax
from jax.experimental import pallas as pl
from jax.experimental.pallas import tpu as pltpu
from jax.experimental.pallas import tpu_sc as plsc

_NC = 2
_NS = 16
_NW = _NC * _NS
_F = 4
_CE = 2000
_UN = 5


def _mesh():
    return plsc.VectorSubcoreMesh(
        core_axis_name="c", subcore_axis_name="s",
        num_cores=_NC, num_subcores=_NS)


def _sc_degree(dst_flat, ew_flat, n_nodes):
    e_total = dst_flat.shape[0]
    epw = e_total // _NW
    nvec = epw // 16
    nz = n_nodes // 16

    @functools.partial(
        pl.kernel,
        out_type=jax.ShapeDtypeStruct((_NW, n_nodes), jnp.float32),
        mesh=_mesh(),
        compiler_params=pltpu.CompilerParams(needs_layout_passes=False),
        scratch_types=[
            pltpu.VMEM((epw,), jnp.int32),
            pltpu.VMEM((epw,), jnp.float32),
            pltpu.VMEM((n_nodes,), jnp.float32),
        ],
    )
    def deg_kernel(dst_hbm, ew_hbm, out_hbm, idx_v, w_v, deg_v):
        c = lax.axis_index("c")
        s = lax.axis_index("s")
        t = c * _NS + s
        pltpu.sync_copy(dst_hbm.at[pl.ds(t * epw, epw)], idx_v)
        pltpu.sync_copy(ew_hbm.at[pl.ds(t * epw, epw)], w_v)
        zeros = jnp.zeros((16,), jnp.float32)

        def zbody(i, carry):
            deg_v[pl.ds(i * 16, 16)] = zeros
            return carry
        lax.fori_loop(0, nz, zbody, 0)

        @plsc.parallel_loop(0, nvec, step=1, unroll=5)
        def _(i):
            idx = idx_v[pl.ds(i * 16, 16)]
            w = w_v[pl.ds(i * 16, 16)]
            plsc.addupdate_scatter(deg_v, [idx], w)
        pltpu.sync_copy(deg_v, out_hbm.at[t])

    return deg_kernel(dst_flat, ew_flat)


def _sc_propagate(table_t, epk, ew, n_nodes):
    h = table_t.shape[0]
    e_total = epk.shape[0]
    owners = h // _F
    es = _NS // owners
    npart = _NC * es
    ept = e_total // npart
    nchunk = ept // _CE
    nvec = _CE // 16
    nz = n_nodes // 16
    assert ept % _CE == 0 and _NS % owners == 0

    @functools.partial(
        pl.kernel,
        out_type=jax.ShapeDtypeStruct((npart, h, n_nodes), jnp.float32),
        mesh=_mesh(),
        compiler_params=pltpu.CompilerParams(needs_layout_passes=False,
                                             use_tc_tiling_on_sc=False),
        scratch_types=(
            [pltpu.VMEM((n_nodes,), jnp.float32) for _ in range(2 * _F)]
            + [pltpu.VMEM((2 * _CE,), jnp.int32),
               pltpu.VMEM((2 * _CE,), jnp.float32),
               pltpu.SemaphoreType.DMA]
        ),
    )
    def prop_kernel(tbl_hbm, epk_hbm, ew_hbm, out_hbm, *rest):
        tbl = rest[:_F]
        acc = rest[_F:2 * _F]
        epk_v = rest[2 * _F]
        ew_v = rest[2 * _F + 1]
        sem = rest[2 * _F + 2]
        c = lax.axis_index("c")
        s = lax.axis_index("s")
        o = lax.rem(s, owners)
        sub = lax.div(s, owners)
        part = c * es + sub
        fo = o * _F
        eb = part * ept

        def stage_start(k):
            off = eb + k * _CE
            slot = lax.rem(k, 2) * _CE
            pltpu.async_copy(epk_hbm.at[pl.ds(off, _CE)],
                             epk_v.at[pl.ds(slot, _CE)], sem)
            pltpu.async_copy(ew_hbm.at[pl.ds(off, _CE)],
                             ew_v.at[pl.ds(slot, _CE)], sem)

        def stage_wait(k):
            off = eb + k * _CE
            slot = lax.rem(k, 2) * _CE
            pltpu.make_async_copy(epk_hbm.at[pl.ds(off, _CE)],
                                  epk_v.at[pl.ds(slot, _CE)], sem).wait()
            pltpu.make_async_copy(ew_hbm.at[pl.ds(off, _CE)],
                                  ew_v.at[pl.ds(slot, _CE)], sem).wait()

        stage_start(0)
        for f in range(_F):
            pltpu.sync_copy(tbl_hbm.at[fo + f], tbl[f])
        zeros = jnp.zeros((16,), jnp.float32)

        def zbody(i, carry):
            for f in range(_F):
                acc[f][pl.ds(i * 16, 16)] = zeros
            return carry
        lax.fori_loop(0, nz, zbody, 0)

        def chunk(k, carry):
            stage_wait(k)

            @pl.when(k + 1 < nchunk)
            def _():
                stage_start(k + 1)
            base = lax.rem(k, 2) * _CE

            @plsc.parallel_loop(0, nvec, step=1, unroll=_UN)
            def _(v):
                off = base + v * 16
                pk = epk_v[pl.ds(off, 16)]
                w = ew_v[pl.ds(off, 16)]
                src16 = lax.bitwise_and(pk, 0xFFFF)
                dst16 = lax.shift_right_logical(pk, 16)
                for f in range(_F):
                    vals = plsc.load_gather(tbl[f], [src16])
                    plsc.addupdate_scatter(acc[f], [dst16], vals * w)
            return carry
        lax.fori_loop(0, nchunk, chunk, 0)

        for f in range(_F):
            pltpu.sync_copy(acc[f], out_hbm.at[part, fo + f])

    return prop_kernel(table_t, epk, ew)


def _tc_mm(x, w1, edge_index):
    n = x.shape[0]
    h1 = w1.shape[1]
    e_total = edge_index.shape[1]

    def body(x_ref, w_ref, ei_ref, s_ref, epk_ref):
        s_ref[...] = lax.dot_general(w_ref[...], x_ref[...],
                                     (((0,), (1,)), ((), ())),
                                     preferred_element_type=jnp.float32)
        epk_ref[...] = jnp.bitwise_or(ei_ref[0],
                                      jnp.left_shift(ei_ref[1], 16))

    return pl.pallas_call(
        body,
        out_shape=(jax.ShapeDtypeStruct((h1, n), jnp.float32),
                   jax.ShapeDtypeStruct((e_total,), jnp.int32)),
    )(x, w1, edge_index)


def _tc_scale(mmt, degp):
    h1, n = mmt.shape

    def body(m_ref, dp_ref, s_ref, dinv_ref):
        deg = jnp.sum(dp_ref[...], axis=0) + 1.0
        dinv = lax.rsqrt(deg)[None, :]
        dinv_ref[...] = dinv
        s_ref[...] = m_ref[...] * dinv

    return pl.pallas_call(
        body,
        out_shape=(jax.ShapeDtypeStruct((h1, n), jnp.float32),
                   jax.ShapeDtypeStruct((1, n), jnp.float32)),
    )(mmt, degp)


def _tc_mid(acc_t, s1_t, dinv, b1, w2):
    h2 = w2.shape[1]
    n = s1_t.shape[1]

    def body(a_ref, s_ref, di_ref, b_ref, w_ref, o_ref):
        hpre = (jnp.sum(a_ref[...], axis=0) + s_ref[...]) * di_ref[...]
        hh = jnp.maximum(hpre + b_ref[...][:, None], 0.0)
        st = lax.dot_general(w_ref[...], hh, (((0,), (0,)), ((), ())),
                             preferred_element_type=jnp.float32)
        o_ref[...] = st * di_ref[...]

    return pl.pallas_call(
        body,
        out_shape=jax.ShapeDtypeStruct((h2, n), jnp.float32),
    )(acc_t, s1_t, dinv, b1, w2)


def _tc_final(acc_t, s2_t, dinv, b2, wl, bl):
    n = s2_t.shape[1]

    def body(a_ref, s_ref, di_ref, b_ref, wl_ref, bl_ref, o_ref):
        hpre = (jnp.sum(a_ref[...], axis=0) + s_ref[...]) * di_ref[...]
        hh = jnp.maximum(hpre + b_ref[...][:, None], 0.0)
        ot = lax.dot_general(wl_ref[...], hh, (((0,), (0,)), ((), ())),
                             preferred_element_type=jnp.float32)
        o_ref[...] = ot + bl_ref[...][:, None]

    return pl.pallas_call(
        body,
        out_shape=jax.ShapeDtypeStruct((1, n), jnp.float32),
    )(acc_t, s2_t, dinv, b2, wl, bl)


def kernel(x, edge_index, edge_weight, W1, b1, W2, b2, Wl, bl):
    n = x.shape[0]
    dst = edge_index[1]

    degp = _sc_degree(dst, edge_weight, n)
    mmt, epk = _tc_mm(x, W1, edge_index)
    s1t, dinv = _tc_scale(mmt, degp)
    acc1 = _sc_propagate(s1t, epk, edge_weight, n)
    s2t = _tc_mid(acc1, s1t, dinv, b1, W2)
    acc2 = _sc_propagate(s2t, epk, edge_weight, n)
    out = _tc_final(acc2, s2t, dinv, b2, Wl, bl)
    return out[0]
